# Initial kernel scaffold; baseline (speedup 1.0000x reference)
import jax
import jax.numpy as jnp
from jax.experimental import pallas as pl


def kernel(x, W, b):
    raise NotImplementedError("write your pallas kernel here")


# speedup vs baseline: 7.4946x; 7.4946x over previous
import functools

import jax
import jax.numpy as jnp
from jax.experimental import pallas as pl
from jax.experimental.pallas import tpu as pltpu

D_MODEL = 768
NUM_EXPERTS = 64
TOP_K = 8
BLOCK_R = 1024


def _router_body(x_ref, w_ref, b_ref, logits_ref, rw_ref):
    l = jnp.dot(x_ref[...], w_ref[...],
                preferred_element_type=jnp.float32) + b_ref[...]
    logits_ref[...] = l

    idx = jax.lax.broadcasted_iota(jnp.int32, l.shape, 1)
    m = l
    for _ in range(TOP_K):
        mx = jnp.max(m, axis=1, keepdims=True)
        eq = m == mx
        fi = jnp.min(jnp.where(eq, idx, NUM_EXPERTS), axis=1, keepdims=True)
        m = jnp.where(eq & (idx == fi), -jnp.inf, m)

    sel = m != l
    q = jnp.minimum(jnp.floor(jnp.maximum(l, 0.0) * 2.0) * 0.5, 7.5)
    rw_ref[...] = jnp.where(sel, q, 0.0)


@functools.partial(jax.jit, static_argnames=())
def kernel(x, W, b):
    n_tokens = x.shape[0]
    grid = (n_tokens // BLOCK_R,)
    logits, rw = pl.pallas_call(
        _router_body,
        grid=grid,
        in_specs=[
            pl.BlockSpec((BLOCK_R, D_MODEL), lambda i: (i, 0)),
            pl.BlockSpec((D_MODEL, NUM_EXPERTS), lambda i: (0, 0)),
            pl.BlockSpec((1, NUM_EXPERTS), lambda i: (0, 0)),
        ],
        out_specs=[
            pl.BlockSpec((BLOCK_R, NUM_EXPERTS), lambda i: (i, 0)),
            pl.BlockSpec((BLOCK_R, NUM_EXPERTS), lambda i: (i, 0)),
        ],
        out_shape=[
            jax.ShapeDtypeStruct((n_tokens, NUM_EXPERTS), jnp.float32),
            jax.ShapeDtypeStruct((n_tokens, NUM_EXPERTS), jnp.float32),
        ],
        compiler_params=pltpu.CompilerParams(
            dimension_semantics=("arbitrary",),
        ),
    )(x, W, b.reshape(1, NUM_EXPERTS))
    return (rw, logits)


# speedup vs baseline: 13.1535x; 1.7551x over previous
import functools

import jax
import jax.numpy as jnp
from jax.experimental import pallas as pl
from jax.experimental.pallas import tpu as pltpu

D_MODEL = 768
NUM_EXPERTS = 64
TOP_K = 8
BLOCK_R = 1024


def _router_body(x_ref, w_ref, b_ref, logits_ref, rw_ref):
    lt = jax.lax.dot_general(
        w_ref[...], x_ref[...],
        dimension_numbers=(((0,), (1,)), ((), ())),
        preferred_element_type=jnp.float32,
    ) + b_ref[...]

    idx = jax.lax.broadcasted_iota(jnp.int32, lt.shape, 0)
    m = lt
    for _ in range(TOP_K):
        mx = jnp.max(m, axis=0, keepdims=True)
        eq = m == mx
        fi = jnp.min(jnp.where(eq, idx, NUM_EXPERTS), axis=0, keepdims=True)
        m = jnp.where(eq & (idx == fi), -jnp.inf, m)

    sel = m != lt
    q = jnp.minimum(jnp.floor(jnp.maximum(lt, 0.0) * 2.0) * 0.5, 7.5)
    rwt = jnp.where(sel, q, 0.0)
    logits_ref[...] = lt.T
    rw_ref[...] = rwt.T


@functools.partial(jax.jit, static_argnames=())
def kernel(x, W, b):
    n_tokens = x.shape[0]
    grid = (n_tokens // BLOCK_R,)
    logits, rw = pl.pallas_call(
        _router_body,
        grid=grid,
        in_specs=[
            pl.BlockSpec((BLOCK_R, D_MODEL), lambda i: (i, 0)),
            pl.BlockSpec((D_MODEL, NUM_EXPERTS), lambda i: (0, 0)),
            pl.BlockSpec((NUM_EXPERTS, 1), lambda i: (0, 0)),
        ],
        out_specs=[
            pl.BlockSpec((BLOCK_R, NUM_EXPERTS), lambda i: (i, 0)),
            pl.BlockSpec((BLOCK_R, NUM_EXPERTS), lambda i: (i, 0)),
        ],
        out_shape=[
            jax.ShapeDtypeStruct((n_tokens, NUM_EXPERTS), jnp.float32),
            jax.ShapeDtypeStruct((n_tokens, NUM_EXPERTS), jnp.float32),
        ],
        compiler_params=pltpu.CompilerParams(
            dimension_semantics=("arbitrary",),
        ),
    )(x, W, b.reshape(NUM_EXPERTS, 1))
    return (rw, logits)


# speedup vs baseline: 15.0956x; 1.1476x over previous
import functools

import jax
import jax.numpy as jnp
from jax.experimental import pallas as pl
from jax.experimental.pallas import tpu as pltpu

D_MODEL = 768
NUM_EXPERTS = 64
TOP_K = 8
BLOCK_R = 2048


def _router_body(x_ref, w_ref, b_ref, logits_ref, rw_ref):
    lt = jax.lax.dot_general(
        w_ref[...], x_ref[...],
        dimension_numbers=(((0,), (1,)), ((), ())),
        preferred_element_type=jnp.float32,
    ) + b_ref[...]

    idx = jax.lax.broadcasted_iota(jnp.int32, lt.shape, 0)
    m = lt
    for _ in range(TOP_K):
        mx = jnp.max(m, axis=0, keepdims=True)
        eq = m == mx
        fi = jnp.min(jnp.where(eq, idx, NUM_EXPERTS), axis=0, keepdims=True)
        m = jnp.where(eq & (idx == fi), -jnp.inf, m)

    sel = m != lt
    q = jnp.minimum(jnp.floor(jnp.maximum(lt, 0.0) * 2.0) * 0.5, 7.5)
    rwt = jnp.where(sel, q, 0.0)
    logits_ref[...] = lt.T
    rw_ref[...] = rwt.T


@functools.partial(jax.jit, static_argnames=())
def kernel(x, W, b):
    n_tokens = x.shape[0]
    grid = (n_tokens // BLOCK_R,)
    logits, rw = pl.pallas_call(
        _router_body,
        grid=grid,
        in_specs=[
            pl.BlockSpec((BLOCK_R, D_MODEL), lambda i: (i, 0)),
            pl.BlockSpec((D_MODEL, NUM_EXPERTS), lambda i: (0, 0)),
            pl.BlockSpec((NUM_EXPERTS, 1), lambda i: (0, 0)),
        ],
        out_specs=[
            pl.BlockSpec((BLOCK_R, NUM_EXPERTS), lambda i: (i, 0)),
            pl.BlockSpec((BLOCK_R, NUM_EXPERTS), lambda i: (i, 0)),
        ],
        out_shape=[
            jax.ShapeDtypeStruct((n_tokens, NUM_EXPERTS), jnp.float32),
            jax.ShapeDtypeStruct((n_tokens, NUM_EXPERTS), jnp.float32),
        ],
        compiler_params=pltpu.CompilerParams(
            dimension_semantics=("arbitrary",),
        ),
    )(x, W, b.reshape(NUM_EXPERTS, 1))
    return (rw, logits)


# speedup vs baseline: 15.7046x; 1.0403x over previous
import functools

import jax
import jax.numpy as jnp
from jax.experimental import pallas as pl
from jax.experimental.pallas import tpu as pltpu

D_MODEL = 768
NUM_EXPERTS = 64
TOP_K = 8
BLOCK_R = 4096


def _router_body(x_ref, w_ref, b_ref, logits_ref, rw_ref):
    lt = jax.lax.dot_general(
        w_ref[...], x_ref[...],
        dimension_numbers=(((0,), (1,)), ((), ())),
        preferred_element_type=jnp.float32,
    ) + b_ref[...]

    idx = jax.lax.broadcasted_iota(jnp.int32, lt.shape, 0)
    m = lt
    for _ in range(TOP_K):
        mx = jnp.max(m, axis=0, keepdims=True)
        eq = m == mx
        fi = jnp.min(jnp.where(eq, idx, NUM_EXPERTS), axis=0, keepdims=True)
        m = jnp.where(eq & (idx == fi), -jnp.inf, m)

    sel = m != lt
    q = jnp.minimum(jnp.floor(jnp.maximum(lt, 0.0) * 2.0) * 0.5, 7.5)
    rwt = jnp.where(sel, q, 0.0)
    logits_ref[...] = lt.T
    rw_ref[...] = rwt.T


@functools.partial(jax.jit, static_argnames=())
def kernel(x, W, b):
    n_tokens = x.shape[0]
    grid = (n_tokens // BLOCK_R,)
    logits, rw = pl.pallas_call(
        _router_body,
        grid=grid,
        in_specs=[
            pl.BlockSpec((BLOCK_R, D_MODEL), lambda i: (i, 0)),
            pl.BlockSpec((D_MODEL, NUM_EXPERTS), lambda i: (0, 0)),
            pl.BlockSpec((NUM_EXPERTS, 1), lambda i: (0, 0)),
        ],
        out_specs=[
            pl.BlockSpec((BLOCK_R, NUM_EXPERTS), lambda i: (i, 0)),
            pl.BlockSpec((BLOCK_R, NUM_EXPERTS), lambda i: (i, 0)),
        ],
        out_shape=[
            jax.ShapeDtypeStruct((n_tokens, NUM_EXPERTS), jnp.float32),
            jax.ShapeDtypeStruct((n_tokens, NUM_EXPERTS), jnp.float32),
        ],
        compiler_params=pltpu.CompilerParams(
            dimension_semantics=("arbitrary",),
        ),
    )(x, W, b.reshape(NUM_EXPERTS, 1))
    return (rw, logits)
